# Initial kernel scaffold; baseline (speedup 1.0000x reference)
import jax
import jax.numpy as jnp
from jax.experimental import pallas as pl


def kernel(x, edge_index, W1_l, b1_l, W1_r, W2_l, b2_l, W2_r, W_h, b_h):
    raise NotImplementedError("write your pallas kernel here")


# speedup vs baseline: 10.0480x; 10.0480x over previous
import functools

import jax
import jax.numpy as jnp
from jax import lax
from jax.experimental import pallas as pl
from jax.experimental.pallas import tpu as pltpu
from jax.experimental.pallas import tpu_sc as plsc

N_NODES = 10000
N_EDGES = 320000
IN_DIM = 128
HID = 32
OUT = 3

NC = 2
NS = 16
NW = NC * NS
CHUNK = 128
CHUNKS = 79
E_PAD = NW * CHUNKS * CHUNK
N_PAD = 10240
ROWS_PER_SUB = N_PAD // NS
TC_BLK = 1024
TC_GRID = N_PAD // TC_BLK


def _make_sc_agg(with_deg: bool):
    mesh = plsc.VectorSubcoreMesh(core_axis_name="c", subcore_axis_name="s")
    out_type = [jax.ShapeDtypeStruct((NC, N_PAD, HID), jnp.float32)]
    scratch = [
        pltpu.VMEM((CHUNKS, CHUNK), jnp.int32),
        pltpu.VMEM((CHUNKS, CHUNK), jnp.int32),
        pltpu.VMEM((CHUNK, HID), jnp.float32),
        pltpu.VMEM_SHARED((N_PAD, HID), jnp.float32),
        pltpu.SemaphoreType.DMA,
    ]
    if with_deg:
        out_type.append(jax.ShapeDtypeStruct((NC, N_PAD, HID), jnp.float32))
        scratch += [
            pltpu.VMEM((CHUNK, HID), jnp.float32),
            pltpu.VMEM_SHARED((N_PAD, HID), jnp.float32),
        ]

    @functools.partial(pl.kernel, out_type=out_type, mesh=mesh,
                       scratch_types=scratch,
                       compiler_params=pltpu.CompilerParams(use_tc_tiling_on_sc=False))
    def agg(table, src3, dst3, zeros_hbm, ones_hbm, *rest):
        if with_deg:
            acc_out, deg_out, src_v, dst_v, rows_v, acc_sh, sem, ones_v, deg_sh = rest
        else:
            acc_out, src_v, dst_v, rows_v, acc_sh, sem = rest
        c = lax.axis_index("c")
        s = lax.axis_index("s")
        wid = c * NS + s
        pltpu.sync_copy(zeros_hbm, acc_sh.at[pl.ds(s * ROWS_PER_SUB, ROWS_PER_SUB)])
        if with_deg:
            pltpu.sync_copy(zeros_hbm, deg_sh.at[pl.ds(s * ROWS_PER_SUB, ROWS_PER_SUB)])
            pltpu.sync_copy(ones_hbm, ones_v)
        pltpu.sync_copy(src3.at[wid], src_v)
        pltpu.sync_copy(dst3.at[wid], dst_v)
        plsc.subcore_barrier()

        def step(j, carry):
            pltpu.async_copy(table.at[src_v.at[j]], rows_v, sem).wait()
            pltpu.sync_copy(rows_v, acc_sh.at[dst_v.at[j]], add=True)
            if with_deg:
                pltpu.sync_copy(ones_v, deg_sh.at[dst_v.at[j]], add=True)
            return carry

        lax.fori_loop(0, CHUNKS, step, 0)
        plsc.subcore_barrier()
        r0 = s * ROWS_PER_SUB
        pltpu.sync_copy(acc_sh.at[pl.ds(r0, ROWS_PER_SUB)],
                        acc_out.at[c, pl.ds(r0, ROWS_PER_SUB)])
        if with_deg:
            pltpu.sync_copy(deg_sh.at[pl.ds(r0, ROWS_PER_SUB)],
                            deg_out.at[c, pl.ds(r0, ROWS_PER_SUB)])

    return agg


_sc_agg_deg = _make_sc_agg(True)
_sc_agg = _make_sc_agg(False)


def _dotT(a, w):
    return lax.dot_general(a, w, (((1,), (1,)), ((), ())),
                           preferred_element_type=jnp.float32)


def _tc_pre(x_pad, W1_l, W1_r, b1_l):
    def body(x_ref, wl_ref, wr_ref, b_ref, xl_ref, p1_ref):
        xb = x_ref[...]
        xl_ref[...] = _dotT(xb, wl_ref[...])
        p1_ref[...] = _dotT(xb, wr_ref[...]) + b_ref[...]

    return pl.pallas_call(
        body,
        grid=(TC_GRID,),
        in_specs=[
            pl.BlockSpec((TC_BLK, IN_DIM), lambda i: (i, 0)),
            pl.BlockSpec((HID, IN_DIM), lambda i: (0, 0)),
            pl.BlockSpec((HID, IN_DIM), lambda i: (0, 0)),
            pl.BlockSpec((1, HID), lambda i: (0, 0)),
        ],
        out_specs=[pl.BlockSpec((TC_BLK, HID), lambda i: (i, 0))] * 2,
        out_shape=[jax.ShapeDtypeStruct((N_PAD, HID), jnp.float32)] * 2,
    )(x_pad, W1_l, W1_r, b1_l.reshape(1, HID))


def _tc_mid(acc, deg, pre1, W2_l, W2_r, b2_l):
    def body(acc_ref, deg_ref, p1_ref, wl_ref, wr_ref, b_ref, hl_ref, p2_ref):
        agg = acc_ref[0] + acc_ref[1]
        degf = jnp.maximum(deg_ref[0] + deg_ref[1], 1.0)
        h = jnp.maximum(agg / degf + p1_ref[...], 0.0)
        hl_ref[...] = _dotT(h, wl_ref[...])
        p2_ref[...] = _dotT(h, wr_ref[...]) + b_ref[...]

    return pl.pallas_call(
        body,
        grid=(TC_GRID,),
        in_specs=[
            pl.BlockSpec((NC, TC_BLK, HID), lambda i: (0, i, 0)),
            pl.BlockSpec((NC, TC_BLK, HID), lambda i: (0, i, 0)),
            pl.BlockSpec((TC_BLK, HID), lambda i: (i, 0)),
            pl.BlockSpec((HID, HID), lambda i: (0, 0)),
            pl.BlockSpec((HID, HID), lambda i: (0, 0)),
            pl.BlockSpec((1, HID), lambda i: (0, 0)),
        ],
        out_specs=[pl.BlockSpec((TC_BLK, HID), lambda i: (i, 0))] * 2,
        out_shape=[jax.ShapeDtypeStruct((N_PAD, HID), jnp.float32)] * 2,
    )(acc, deg, pre1, W2_l, W2_r, b2_l.reshape(1, HID))


def _tc_out(acc2, deg, pre2, W_h, b_h):
    def body(acc_ref, deg_ref, p2_ref, wh_ref, bh_ref, out_ref):
        agg = acc_ref[0] + acc_ref[1]
        degf = jnp.maximum(deg_ref[0] + deg_ref[1], 1.0)
        h2 = jnp.maximum(agg / degf + p2_ref[...], 0.0)
        out_ref[...] = _dotT(h2, wh_ref[...]) + bh_ref[...]

    return pl.pallas_call(
        body,
        grid=(TC_GRID,),
        in_specs=[
            pl.BlockSpec((NC, TC_BLK, HID), lambda i: (0, i, 0)),
            pl.BlockSpec((NC, TC_BLK, HID), lambda i: (0, i, 0)),
            pl.BlockSpec((TC_BLK, HID), lambda i: (i, 0)),
            pl.BlockSpec((OUT, HID), lambda i: (0, 0)),
            pl.BlockSpec((1, OUT), lambda i: (0, 0)),
        ],
        out_specs=pl.BlockSpec((TC_BLK, OUT), lambda i: (i, 0)),
        out_shape=jax.ShapeDtypeStruct((N_PAD, OUT), jnp.float32),
    )(acc2, deg, pre2, W_h, b_h.reshape(1, OUT))


def kernel(x, edge_index, W1_l, b1_l, W1_r, W2_l, b2_l, W2_r, W_h, b_h):
    src = edge_index[0].astype(jnp.int32)
    dst = edge_index[1].astype(jnp.int32)
    npad = E_PAD - N_EDGES
    src3 = jnp.concatenate([src, jnp.zeros((npad,), jnp.int32)]).reshape(NW, CHUNKS, CHUNK)
    dst3 = jnp.concatenate([dst, jnp.full((npad,), N_NODES, jnp.int32)]).reshape(NW, CHUNKS, CHUNK)
    x_pad = jnp.pad(x, ((0, N_PAD - N_NODES), (0, 0)))
    zeros_hbm = jnp.zeros((ROWS_PER_SUB, HID), jnp.float32)
    ones_hbm = jnp.ones((CHUNK, HID), jnp.float32)

    xl, pre1 = _tc_pre(x_pad, W1_l, W1_r, b1_l)
    acc1, deg = _sc_agg_deg(xl, src3, dst3, zeros_hbm, ones_hbm)
    hl, pre2 = _tc_mid(acc1, deg, pre1, W2_l, W2_r, b2_l)
    (acc2,) = _sc_agg(hl, src3, dst3, zeros_hbm, ones_hbm)
    out_pad = _tc_out(acc2, deg, pre2, W_h, b_h)
    return out_pad[:N_NODES]


# speedup vs baseline: 10.1833x; 1.0135x over previous
import functools

import jax
import jax.numpy as jnp
from jax import lax
from jax.experimental import pallas as pl
from jax.experimental.pallas import tpu as pltpu
from jax.experimental.pallas import tpu_sc as plsc

N_NODES = 10000
N_EDGES = 320000
IN_DIM = 128
HID = 32
OUT = 3

NC = 2
NS = 16
NW = NC * NS
CHUNK = 128
CHUNKS = 80
NBUF = 4
E_PAD = NW * CHUNKS * CHUNK
N_PAD = 10240
ROWS_PER_SUB = N_PAD // NS
TC_BLK = 1024
TC_GRID = N_PAD // TC_BLK


def _make_sc_agg(with_deg: bool):
    mesh = plsc.VectorSubcoreMesh(core_axis_name="c", subcore_axis_name="s")
    out_type = [jax.ShapeDtypeStruct((NC, N_PAD, HID), jnp.float32)]
    scratch = [
        pltpu.VMEM((CHUNKS, CHUNK), jnp.int32),
        pltpu.VMEM((CHUNKS, CHUNK), jnp.int32),
        [pltpu.VMEM((CHUNK, HID), jnp.float32) for _ in range(NBUF)],
        pltpu.VMEM_SHARED((N_PAD, HID), jnp.float32),
        [pltpu.SemaphoreType.DMA for _ in range(NBUF)],
    ]
    if with_deg:
        out_type.append(jax.ShapeDtypeStruct((NC, N_PAD, HID), jnp.float32))
        scratch += [
            pltpu.VMEM((CHUNK, HID), jnp.float32),
            pltpu.VMEM_SHARED((N_PAD, HID), jnp.float32),
        ]

    @functools.partial(pl.kernel, out_type=out_type, mesh=mesh,
                       scratch_types=scratch,
                       compiler_params=pltpu.CompilerParams(use_tc_tiling_on_sc=False))
    def agg(table, src3, dst3, zeros_hbm, ones_hbm, *rest):
        if with_deg:
            acc_out, deg_out, src_v, dst_v, rows, acc_sh, sems, ones_v, deg_sh = rest
        else:
            acc_out, src_v, dst_v, rows, acc_sh, sems = rest
        c = lax.axis_index("c")
        s = lax.axis_index("s")
        wid = c * NS + s
        pltpu.sync_copy(zeros_hbm, acc_sh.at[pl.ds(s * ROWS_PER_SUB, ROWS_PER_SUB)])
        if with_deg:
            pltpu.sync_copy(zeros_hbm, deg_sh.at[pl.ds(s * ROWS_PER_SUB, ROWS_PER_SUB)])
            pltpu.sync_copy(ones_hbm, ones_v)
        pltpu.sync_copy(src3.at[wid], src_v)
        pltpu.sync_copy(dst3.at[wid], dst_v)
        plsc.subcore_barrier()

        for b in range(NBUF):
            pltpu.async_copy(table.at[src_v.at[b]], rows[b], sems[b])

        def step(i, carry):
            for b in range(NBUF):
                j = i * NBUF + b
                pltpu.make_async_copy(table.at[src_v.at[j]], rows[b], sems[b]).wait()
                pltpu.sync_copy(rows[b], acc_sh.at[dst_v.at[j]], add=True)
                if with_deg:
                    pltpu.sync_copy(ones_v, deg_sh.at[dst_v.at[j]], add=True)
                jn = lax.min(j + NBUF, CHUNKS - 1)
                pltpu.async_copy(table.at[src_v.at[jn]], rows[b], sems[b])
            return carry

        lax.fori_loop(0, CHUNKS // NBUF, step, 0)
        for b in range(NBUF):
            pltpu.make_async_copy(table.at[src_v.at[0]], rows[b], sems[b]).wait()
        plsc.subcore_barrier()
        r0 = s * ROWS_PER_SUB
        pltpu.sync_copy(acc_sh.at[pl.ds(r0, ROWS_PER_SUB)],
                        acc_out.at[c, pl.ds(r0, ROWS_PER_SUB)])
        if with_deg:
            pltpu.sync_copy(deg_sh.at[pl.ds(r0, ROWS_PER_SUB)],
                            deg_out.at[c, pl.ds(r0, ROWS_PER_SUB)])

    return agg


_sc_agg_deg = _make_sc_agg(True)
_sc_agg = _make_sc_agg(False)


def _dotT(a, w):
    return lax.dot_general(a, w, (((1,), (1,)), ((), ())),
                           preferred_element_type=jnp.float32)


def _tc_pre(x_pad, W1_l, W1_r, b1_l):
    def body(x_ref, wl_ref, wr_ref, b_ref, xl_ref, p1_ref):
        xb = x_ref[...]
        xl_ref[...] = _dotT(xb, wl_ref[...])
        p1_ref[...] = _dotT(xb, wr_ref[...]) + b_ref[...]

    return pl.pallas_call(
        body,
        grid=(TC_GRID,),
        in_specs=[
            pl.BlockSpec((TC_BLK, IN_DIM), lambda i: (i, 0)),
            pl.BlockSpec((HID, IN_DIM), lambda i: (0, 0)),
            pl.BlockSpec((HID, IN_DIM), lambda i: (0, 0)),
            pl.BlockSpec((1, HID), lambda i: (0, 0)),
        ],
        out_specs=[pl.BlockSpec((TC_BLK, HID), lambda i: (i, 0))] * 2,
        out_shape=[jax.ShapeDtypeStruct((N_PAD, HID), jnp.float32)] * 2,
    )(x_pad, W1_l, W1_r, b1_l.reshape(1, HID))


def _tc_mid(acc, deg, pre1, W2_l, W2_r, b2_l):
    def body(acc_ref, deg_ref, p1_ref, wl_ref, wr_ref, b_ref, hl_ref, p2_ref):
        agg = acc_ref[0] + acc_ref[1]
        degf = jnp.maximum(deg_ref[0] + deg_ref[1], 1.0)
        h = jnp.maximum(agg / degf + p1_ref[...], 0.0)
        hl_ref[...] = _dotT(h, wl_ref[...])
        p2_ref[...] = _dotT(h, wr_ref[...]) + b_ref[...]

    return pl.pallas_call(
        body,
        grid=(TC_GRID,),
        in_specs=[
            pl.BlockSpec((NC, TC_BLK, HID), lambda i: (0, i, 0)),
            pl.BlockSpec((NC, TC_BLK, HID), lambda i: (0, i, 0)),
            pl.BlockSpec((TC_BLK, HID), lambda i: (i, 0)),
            pl.BlockSpec((HID, HID), lambda i: (0, 0)),
            pl.BlockSpec((HID, HID), lambda i: (0, 0)),
            pl.BlockSpec((1, HID), lambda i: (0, 0)),
        ],
        out_specs=[pl.BlockSpec((TC_BLK, HID), lambda i: (i, 0))] * 2,
        out_shape=[jax.ShapeDtypeStruct((N_PAD, HID), jnp.float32)] * 2,
    )(acc, deg, pre1, W2_l, W2_r, b2_l.reshape(1, HID))


def _tc_out(acc2, deg, pre2, W_h, b_h):
    def body(acc_ref, deg_ref, p2_ref, wh_ref, bh_ref, out_ref):
        agg = acc_ref[0] + acc_ref[1]
        degf = jnp.maximum(deg_ref[0] + deg_ref[1], 1.0)
        h2 = jnp.maximum(agg / degf + p2_ref[...], 0.0)
        out_ref[...] = _dotT(h2, wh_ref[...]) + bh_ref[...]

    return pl.pallas_call(
        body,
        grid=(TC_GRID,),
        in_specs=[
            pl.BlockSpec((NC, TC_BLK, HID), lambda i: (0, i, 0)),
            pl.BlockSpec((NC, TC_BLK, HID), lambda i: (0, i, 0)),
            pl.BlockSpec((TC_BLK, HID), lambda i: (i, 0)),
            pl.BlockSpec((OUT, HID), lambda i: (0, 0)),
            pl.BlockSpec((1, OUT), lambda i: (0, 0)),
        ],
        out_specs=pl.BlockSpec((TC_BLK, OUT), lambda i: (i, 0)),
        out_shape=jax.ShapeDtypeStruct((N_PAD, OUT), jnp.float32),
    )(acc2, deg, pre2, W_h, b_h.reshape(1, OUT))


def kernel(x, edge_index, W1_l, b1_l, W1_r, W2_l, b2_l, W2_r, W_h, b_h):
    src = edge_index[0].astype(jnp.int32)
    dst = edge_index[1].astype(jnp.int32)
    npad = E_PAD - N_EDGES
    src3 = jnp.concatenate([src, jnp.zeros((npad,), jnp.int32)]).reshape(NW, CHUNKS, CHUNK)
    dst3 = jnp.concatenate([dst, jnp.full((npad,), N_NODES, jnp.int32)]).reshape(NW, CHUNKS, CHUNK)
    x_pad = jnp.pad(x, ((0, N_PAD - N_NODES), (0, 0)))
    zeros_hbm = jnp.zeros((ROWS_PER_SUB, HID), jnp.float32)
    ones_hbm = jnp.ones((CHUNK, HID), jnp.float32)

    xl, pre1 = _tc_pre(x_pad, W1_l, W1_r, b1_l)
    acc1, deg = _sc_agg_deg(xl, src3, dst3, zeros_hbm, ones_hbm)
    hl, pre2 = _tc_mid(acc1, deg, pre1, W2_l, W2_r, b2_l)
    (acc2,) = _sc_agg(hl, src3, dst3, zeros_hbm, ones_hbm)
    out_pad = _tc_out(acc2, deg, pre2, W_h, b_h)
    return out_pad[:N_NODES]


# speedup vs baseline: 10.6850x; 1.0493x over previous
import functools

import jax
import jax.numpy as jnp
from jax import lax
from jax.experimental import pallas as pl
from jax.experimental.pallas import tpu as pltpu
from jax.experimental.pallas import tpu_sc as plsc

N_NODES = 10000
N_EDGES = 320000
IN_DIM = 128
HID = 32
OUT = 3

NC = 2
NS = 16
NW = NC * NS
CHUNK = 128
CHUNKS = 80
NBUF = 4
E_PAD = NW * CHUNKS * CHUNK
N_PAD = 10240
ROWS_PER_SUB = N_PAD // NS
TC_BLK = 1024
TC_GRID = N_PAD // TC_BLK
DEGW = 16


def _make_sc_agg(with_deg: bool):
    mesh = plsc.VectorSubcoreMesh(core_axis_name="c", subcore_axis_name="s")
    out_type = [jax.ShapeDtypeStruct((NC, N_PAD, HID), jnp.float32)]
    scratch = [
        pltpu.VMEM((CHUNKS, CHUNK), jnp.int32),
        pltpu.VMEM((CHUNKS, CHUNK), jnp.int32),
        [pltpu.VMEM((CHUNK, HID), jnp.float32) for _ in range(NBUF)],
        pltpu.VMEM_SHARED((N_PAD, HID), jnp.float32),
        [pltpu.SemaphoreType.DMA for _ in range(NBUF)],
        [pltpu.SemaphoreType.DMA for _ in range(NBUF)],
    ]
    if with_deg:
        out_type.append(jax.ShapeDtypeStruct((NC, N_PAD, DEGW), jnp.float32))
        scratch += [
            pltpu.VMEM((CHUNK, DEGW), jnp.float32),
            pltpu.VMEM_SHARED((N_PAD, DEGW), jnp.float32),
            [pltpu.SemaphoreType.DMA for _ in range(NBUF)],
        ]

    @functools.partial(pl.kernel, out_type=out_type, mesh=mesh,
                       scratch_types=scratch,
                       compiler_params=pltpu.CompilerParams(use_tc_tiling_on_sc=False))
    def agg(table, src3, dst3, zeros_hbm, zeros16_hbm, ones_hbm, *rest):
        if with_deg:
            (acc_out, deg_out, src_v, dst_v, rows, acc_sh, gsem, ssem,
             ones_v, deg_sh, dsem) = rest
        else:
            acc_out, src_v, dst_v, rows, acc_sh, gsem, ssem = rest
        c = lax.axis_index("c")
        s = lax.axis_index("s")
        wid = c * NS + s
        pltpu.sync_copy(zeros_hbm, acc_sh.at[pl.ds(s * ROWS_PER_SUB, ROWS_PER_SUB)])
        if with_deg:
            pltpu.sync_copy(zeros16_hbm, deg_sh.at[pl.ds(s * ROWS_PER_SUB, ROWS_PER_SUB)])
            pltpu.sync_copy(ones_hbm, ones_v)
        pltpu.sync_copy(src3.at[wid], src_v)
        pltpu.sync_copy(dst3.at[wid], dst_v)
        plsc.subcore_barrier()

        def gat(j, b):
            pltpu.async_copy(table.at[src_v.at[j]], rows[b], gsem[b])

        def scat(j, b):
            pltpu.make_async_copy(table.at[src_v.at[j]], rows[b], gsem[b]).wait()
            pltpu.async_copy(rows[b], acc_sh.at[dst_v.at[j]], ssem[b], add=True)
            if with_deg:
                pltpu.async_copy(ones_v, deg_sh.at[dst_v.at[j]], dsem[b], add=True)

        def wait_scat(b):
            pltpu.make_async_copy(rows[b], acc_sh.at[dst_v.at[0]], ssem[b]).wait()
            if with_deg:
                pltpu.make_async_copy(ones_v, deg_sh.at[dst_v.at[0]], dsem[b]).wait()

        gat(0, 0)
        gat(1, 1)
        gat(2, 2)
        scat(0, 0)
        gat(3, 3)
        scat(1, 1)

        def step(i, carry):
            for b in range(NBUF):
                j = NBUF + i * NBUF + b
                wait_scat(b)
                gat(j, b)
                scat(j - 2, (b + 2) % NBUF)
            return carry

        lax.fori_loop(0, (CHUNKS - NBUF) // NBUF, step, 0)
        scat(CHUNKS - 2, (CHUNKS - 2) % NBUF)
        scat(CHUNKS - 1, (CHUNKS - 1) % NBUF)
        for b in range(NBUF):
            wait_scat(b)
        plsc.subcore_barrier()
        r0 = s * ROWS_PER_SUB
        pltpu.sync_copy(acc_sh.at[pl.ds(r0, ROWS_PER_SUB)],
                        acc_out.at[c, pl.ds(r0, ROWS_PER_SUB)])
        if with_deg:
            pltpu.sync_copy(deg_sh.at[pl.ds(r0, ROWS_PER_SUB)],
                            deg_out.at[c, pl.ds(r0, ROWS_PER_SUB)])

    return agg


_sc_agg_deg = _make_sc_agg(True)
_sc_agg = _make_sc_agg(False)


def _dotT(a, w):
    return lax.dot_general(a, w, (((1,), (1,)), ((), ())),
                           preferred_element_type=jnp.float32)


def _tc_pre(x_pad, W1_l, W1_r, b1_l):
    def body(x_ref, wl_ref, wr_ref, b_ref, xl_ref, p1_ref):
        xb = x_ref[...]
        xl_ref[...] = _dotT(xb, wl_ref[...])
        p1_ref[...] = _dotT(xb, wr_ref[...]) + b_ref[...]

    return pl.pallas_call(
        body,
        grid=(TC_GRID,),
        in_specs=[
            pl.BlockSpec((TC_BLK, IN_DIM), lambda i: (i, 0)),
            pl.BlockSpec((HID, IN_DIM), lambda i: (0, 0)),
            pl.BlockSpec((HID, IN_DIM), lambda i: (0, 0)),
            pl.BlockSpec((1, HID), lambda i: (0, 0)),
        ],
        out_specs=[pl.BlockSpec((TC_BLK, HID), lambda i: (i, 0))] * 2,
        out_shape=[jax.ShapeDtypeStruct((N_PAD, HID), jnp.float32)] * 2,
    )(x_pad, W1_l, W1_r, b1_l.reshape(1, HID))


def _tc_mid(acc, deg, pre1, W2_l, W2_r, b2_l):
    def body(acc_ref, deg_ref, p1_ref, wl_ref, wr_ref, b_ref, hl_ref, p2_ref):
        agg = acc_ref[0] + acc_ref[1]
        degf = jnp.maximum((deg_ref[0] + deg_ref[1])[:, 0:1], 1.0)
        h = jnp.maximum(agg / degf + p1_ref[...], 0.0)
        hl_ref[...] = _dotT(h, wl_ref[...])
        p2_ref[...] = _dotT(h, wr_ref[...]) + b_ref[...]

    return pl.pallas_call(
        body,
        grid=(TC_GRID,),
        in_specs=[
            pl.BlockSpec((NC, TC_BLK, HID), lambda i: (0, i, 0)),
            pl.BlockSpec((NC, TC_BLK, DEGW), lambda i: (0, i, 0)),
            pl.BlockSpec((TC_BLK, HID), lambda i: (i, 0)),
            pl.BlockSpec((HID, HID), lambda i: (0, 0)),
            pl.BlockSpec((HID, HID), lambda i: (0, 0)),
            pl.BlockSpec((1, HID), lambda i: (0, 0)),
        ],
        out_specs=[pl.BlockSpec((TC_BLK, HID), lambda i: (i, 0))] * 2,
        out_shape=[jax.ShapeDtypeStruct((N_PAD, HID), jnp.float32)] * 2,
    )(acc, deg, pre1, W2_l, W2_r, b2_l.reshape(1, HID))


def _tc_out(acc2, deg, pre2, W_h, b_h):
    def body(acc_ref, deg_ref, p2_ref, wh_ref, bh_ref, out_ref):
        agg = acc_ref[0] + acc_ref[1]
        degf = jnp.maximum((deg_ref[0] + deg_ref[1])[:, 0:1], 1.0)
        h2 = jnp.maximum(agg / degf + p2_ref[...], 0.0)
        out_ref[...] = _dotT(h2, wh_ref[...]) + bh_ref[...]

    return pl.pallas_call(
        body,
        grid=(TC_GRID,),
        in_specs=[
            pl.BlockSpec((NC, TC_BLK, HID), lambda i: (0, i, 0)),
            pl.BlockSpec((NC, TC_BLK, DEGW), lambda i: (0, i, 0)),
            pl.BlockSpec((TC_BLK, HID), lambda i: (i, 0)),
            pl.BlockSpec((OUT, HID), lambda i: (0, 0)),
            pl.BlockSpec((1, OUT), lambda i: (0, 0)),
        ],
        out_specs=pl.BlockSpec((TC_BLK, OUT), lambda i: (i, 0)),
        out_shape=jax.ShapeDtypeStruct((N_PAD, OUT), jnp.float32),
    )(acc2, deg, pre2, W_h, b_h.reshape(1, OUT))


def kernel(x, edge_index, W1_l, b1_l, W1_r, W2_l, b2_l, W2_r, W_h, b_h):
    src = edge_index[0].astype(jnp.int32)
    dst = edge_index[1].astype(jnp.int32)
    npad = E_PAD - N_EDGES
    src3 = jnp.concatenate([src, jnp.zeros((npad,), jnp.int32)]).reshape(NW, CHUNKS, CHUNK)
    dst3 = jnp.concatenate([dst, jnp.full((npad,), N_NODES, jnp.int32)]).reshape(NW, CHUNKS, CHUNK)
    x_pad = jnp.pad(x, ((0, N_PAD - N_NODES), (0, 0)))
    zeros_hbm = jnp.zeros((ROWS_PER_SUB, HID), jnp.float32)
    zeros16_hbm = jnp.zeros((ROWS_PER_SUB, DEGW), jnp.float32)
    ones_hbm = jnp.ones((CHUNK, DEGW), jnp.float32)

    xl, pre1 = _tc_pre(x_pad, W1_l, W1_r, b1_l)
    acc1, deg = _sc_agg_deg(xl, src3, dst3, zeros_hbm, zeros16_hbm, ones_hbm)
    hl, pre2 = _tc_mid(acc1, deg, pre1, W2_l, W2_r, b2_l)
    (acc2,) = _sc_agg(hl, src3, dst3, zeros_hbm, zeros16_hbm, ones_hbm)
    out_pad = _tc_out(acc2, deg, pre2, W_h, b_h)
    return out_pad[:N_NODES]
